# baseline (speedup 1.0000x reference)
import functools

import jax
import jax.numpy as jnp
from jax import lax
from jax.experimental import pallas as pl
from jax.experimental.pallas import tpu as pltpu
from jax.experimental.pallas import tpu_sc as plsc

B, S_IN, D = 16384, 66, 128
S0, S_OUT = 42, 24

NUM_CORES = 2
NUM_SUBCORES = 16
NW = NUM_CORES * NUM_SUBCORES
B_PER_W = B // NW
CHUNK = 64
N_CHUNKS = B_PER_W // CHUNK


@functools.partial(
    pl.kernel,
    mesh=plsc.VectorSubcoreMesh(core_axis_name="c", subcore_axis_name="s"),
    out_type=jax.ShapeDtypeStruct((B, S_OUT * D), jnp.float32),
    scratch_types=[pltpu.SemaphoreType.DMA],
)
def _select_lower(in_hbm, out_hbm, sem):
    wid = lax.axis_index("s") * NUM_CORES + lax.axis_index("c")
    base = wid * B_PER_W
    copies = []
    for i in range(N_CHUNKS):
        b0 = base + i * CHUNK
        cp = pltpu.make_async_copy(
            in_hbm.at[pl.ds(b0, CHUNK), pl.ds(S0 * D, S_OUT * D)],
            out_hbm.at[pl.ds(b0, CHUNK)],
            sem,
        )
        cp.start()
        copies.append(cp)
    for cp in copies:
        cp.wait()


def kernel(input):
    flat = input.reshape(B, S_IN * D)
    return _select_lower(flat).reshape(B, S_OUT, D)


# speedup vs baseline: 6.1596x; 6.1596x over previous
import functools

import jax
import jax.numpy as jnp
from jax import lax
from jax.experimental import pallas as pl
from jax.experimental.pallas import tpu as pltpu
from jax.experimental.pallas import tpu_sc as plsc

B, S_IN, D = 16384, 66, 128
S0, S_OUT = 42, 24

NUM_CORES = 2
NUM_SUBCORES = 16
NW = NUM_CORES * NUM_SUBCORES
B_PER_W = B // NW
CHUNK = 16
N_CHUNKS = B_PER_W // CHUNK
NBUF = 2


@functools.partial(
    pl.kernel,
    mesh=plsc.VectorSubcoreMesh(core_axis_name="c", subcore_axis_name="s"),
    out_type=jax.ShapeDtypeStruct((B, S_OUT * D), jnp.float32),
    scratch_types=[
        pltpu.VMEM((NBUF, CHUNK, S_OUT * D), jnp.float32),
        pltpu.SemaphoreType.DMA,
        pltpu.SemaphoreType.DMA,
        pltpu.SemaphoreType.DMA,
        pltpu.SemaphoreType.DMA,
    ],
)
def _select_lower(in_hbm, out_hbm, buf, rs0, rs1, ws0, ws1):
    rsems = [rs0, rs1]
    wsems = [ws0, ws1]
    wid = lax.axis_index("s") * NUM_CORES + lax.axis_index("c")
    base = wid * B_PER_W

    def read(g, slot):
        b0 = base + g * CHUNK
        cp = pltpu.make_async_copy(
            in_hbm.at[pl.ds(b0, CHUNK), pl.ds(S0 * D, S_OUT * D)],
            buf.at[slot],
            rsems[slot],
        )
        cp.start()
        return cp

    def write(g, slot):
        b0 = base + g * CHUNK
        cp = pltpu.make_async_copy(
            buf.at[slot], out_hbm.at[pl.ds(b0, CHUNK)], wsems[slot]
        )
        cp.start()
        return cp

    pending_w = [None] * NBUF
    pending_r = read(0, 0)
    for g in range(N_CHUNKS):
        slot = g % NBUF
        nxt = (g + 1) % NBUF
        if g + 1 < N_CHUNKS:
            if pending_w[nxt] is not None:
                pending_w[nxt].wait()
                pending_w[nxt] = None
            nxt_r = read(g + 1, nxt)
        pending_r.wait()
        pending_w[slot] = write(g, slot)
        if g + 1 < N_CHUNKS:
            pending_r = nxt_r
    for w in pending_w:
        if w is not None:
            w.wait()


def kernel(input):
    flat = input.reshape(B, S_IN * D)
    return _select_lower(flat).reshape(B, S_OUT, D)
